# baseline (speedup 1.0000x reference)
import jax
import jax.numpy as jnp
from jax.experimental import pallas as pl

NN = 128
C = 64
T = 64
B = 16
TOPK = 20
TACT = 16
NROWS = 136
BG_W = 240.0
BN_CNT = 2048.0 * T


def _mm(a, b):
    sh = a.shape
    a2 = a.reshape(-1, sh[-1])
    if a2.dtype != jnp.bfloat16:
        a2 = a2.astype(jnp.bfloat16)
    r = jax.lax.dot_general(a2, b.astype(jnp.bfloat16), (((1,), (0,)), ((), ())),
                            preferred_element_type=jnp.float32)
    return r.reshape(sh[:-1] + (b.shape[-1],))


def _mmt(a, b):
    sh = a.shape
    a2 = a.reshape(-1, sh[-1])
    if a2.dtype != jnp.bfloat16:
        a2 = a2.astype(jnp.bfloat16)
    r = jax.lax.dot_general(a2, b.astype(jnp.bfloat16), (((1,), (1,)), ((), ())),
                            preferred_element_type=jnp.float32)
    return r.reshape(sh[:-1] + (b.shape[0],))


def _tshift(x, s):
    if s == 0:
        return x
    r, t, c = x.shape
    z = jnp.zeros((r, abs(s), c), x.dtype)
    if s > 0:
        return jnp.concatenate([x[:, s:, :], z], axis=1)
    return jnp.concatenate([z, x[:, :t + s, :]], axis=1)


def _body(data_ref, emb_ref, atti_ref, attj_ref, lin_ref, gbias_ref,
          w3t0, w5t0, dwt0, c3b0, c5b0, bn1g0, bn1b0, bn2g0, bn2b0,
          dwb0, pww0, pwb0, bn3g0, bn3b0, fww0, fusb0,
          w3t1, w5t1, dwt1, c3b1, c5b1, bn1g1, bn1b1, bn2g1, bn2b1,
          dwb1, pww1, pwb1, bn3g1, bn3b1, fww1, fusb1,
          bnog_ref, bnob_ref, outw_ref, outb_ref, out_ref):
    f32 = jnp.float32
    emb = emb_ref[...]
    linv = lin_ref[...]

    ebf = emb.astype(jnp.bfloat16)
    g = jax.lax.dot_general(ebf, ebf.T, (((1,), (0,)), ((), ())),
                            preferred_element_type=jnp.float32)
    nrm = jnp.sqrt(jnp.sum(emb * emb, axis=1, keepdims=True))
    cos = g / (nrm * nrm.T)
    mask = jnp.zeros((NN, NN), f32)
    cm = cos
    for _ in range(TOPK):
        rmax = jnp.max(cm, axis=1, keepdims=True)
        hit = cm == rmax
        mask = jnp.where(hit, 1.0, mask)
        cm = jnp.where(hit, -jnp.inf, cm)

    att_i = atti_ref[...]
    att_j = attj_ref[...]
    a_i = jnp.sum(linv * att_i[:, :C])
    a_j = jnp.sum(linv * att_j[:, :C])
    e_i = jnp.sum(emb * att_i[:, C:], axis=1, keepdims=True)
    e_j = jnp.sum(emb * att_j[:, C:], axis=1, keepdims=True)

    xa = data_ref[0, :, 0:TACT]
    xt = xa.T
    alpha = (xt[:, :, None] * a_i + e_i.reshape(1, NN, 1)
             + xt[:, None, :] * a_j + e_j.reshape(1, 1, NN))
    alpha = jnp.where(alpha >= 0, alpha, 0.2 * alpha)
    am = jnp.where(mask[None, :, :] > 0, alpha, -jnp.inf)
    amax = jnp.max(am, axis=2, keepdims=True)
    ex = jnp.exp(am - amax)
    denom = jnp.sum(ex, axis=2, keepdims=True) + 1e-16
    att = ex / denom
    s = jnp.sum(att * xt[:, None, :], axis=2)

    gnn_bias = gbias_ref[...].reshape(1, 1, C)
    st3 = s.T.reshape(NN, TACT, 1)
    x_act = st3 * linv.reshape(1, 1, C) + gnn_bias
    x_all = jnp.concatenate(
        [jnp.concatenate(
            [x_act,
             jnp.broadcast_to(gnn_bias, (NN, T - TACT, C))], axis=1),
         jnp.broadcast_to(gnn_bias, (NROWS - NN, T, C))], axis=0)

    ridx = jax.lax.broadcasted_iota(jnp.int32, (NROWS, 1, 1), 0)
    wr = jnp.where(ridx < NN, 1.0, BG_W)

    def bn(x, gv, bv):
        xw = x * wr
        m = jnp.sum(xw, axis=(0, 1), keepdims=True) / BN_CNT
        msq = jnp.sum(xw * x, axis=(0, 1), keepdims=True) / BN_CNT
        v = msq - m * m
        sc = gv / jnp.sqrt(v + 1e-5)
        return x * sc + (bv - m * sc)

    def row3(ref):
        return ref[...].reshape(1, 1, -1)

    zoo = jnp.zeros((C, C), jnp.bfloat16)

    def ms_block(x, w3t, w5t, dwt, c3b, c5b, bn1g, bn1b, bn2g, bn2b,
                 dwb, pww, pwb, bn3g, bn3b, fww, fusb):
        shf = [_tshift(x, d) for d in (-1, 0, 1)]
        xb = x.astype(jnp.bfloat16)
        xcat = jnp.concatenate(
            [_tshift(xb, d) for d in (-2, -1, 0, 1, 2)], axis=2)
        dwv = dwt[...].T
        res = (shf[0] * dwv[0:1, :].reshape(1, 1, C)
               + shf[1] * dwv[1:2, :].reshape(1, 1, C)
               + shf[2] * dwv[2:3, :].reshape(1, 1, C)
               + row3(dwb))
        res = _mmt(res, pww[...]) + row3(pwb)
        res = bn(res, row3(bn3g), row3(bn3b))
        w3b = jnp.transpose(w3t[...], (2, 1, 0)).astype(jnp.bfloat16)
        w5b = jnp.transpose(w5t[...], (2, 1, 0)).astype(jnp.bfloat16)
        rhs = jnp.concatenate(
            [jnp.concatenate(
                [w3b[dk - 1] if 1 <= dk <= 3 else zoo, w5b[dk]], axis=1)
             for dk in range(5)], axis=0)
        acc = _mm(xcat, rhs)
        acc = acc + jnp.concatenate([row3(c3b), row3(c5b)], axis=2)
        b12 = jax.nn.relu(bn(acc,
                             jnp.concatenate([row3(bn1g), row3(bn2g)], axis=2),
                             jnp.concatenate([row3(bn1b), row3(bn2b)], axis=2)))
        fused = _mmt(b12, fww[...]) + row3(fusb)
        return jax.nn.relu(fused + res)

    h = ms_block(x_all, w3t0, w5t0, dwt0, c3b0, c5b0, bn1g0, bn1b0,
                 bn2g0, bn2b0, dwb0, pww0, pwb0, bn3g0, bn3b0, fww0, fusb0)
    h = ms_block(h, w3t1, w5t1, dwt1, c3b1, c5b1, bn1g1, bn1b1,
                 bn2g1, bn2b1, dwb1, pww1, pwb1, bn3g1, bn3b1, fww1, fusb1)
    hm = jnp.sum(h, axis=1) / float(T)

    y0 = hm[:NN, :] * emb
    ybg = hm[NN:NN + 1, :] * emb
    m = jnp.sum(y0 + 15.0 * ybg, axis=0, keepdims=True) / 2048.0
    d0 = y0 - m
    dbg = ybg - m
    v = (jnp.sum(d0 * d0, axis=0, keepdims=True)
         + 15.0 * jnp.sum(dbg * dbg, axis=0, keepdims=True)) / 2048.0
    bno_g = bnog_ref[...]
    bno_b = bnob_ref[...]
    z0 = jax.nn.relu(d0 / jnp.sqrt(v + 1e-5) * bno_g + bno_b)
    zbg = jax.nn.relu(dbg / jnp.sqrt(v + 1e-5) * bno_g + bno_b)
    outwT = outw_ref[...].T
    row0 = (_mm(z0, outwT) + outb_ref[0, 0]).T
    rbg = (_mm(zbg, outwT) + outb_ref[0, 0]).T
    out_ref[...] = jnp.concatenate(
        [row0, jnp.broadcast_to(rbg, (B - 1, NN))], axis=0)


def kernel(data, params):
    p = params
    f32 = jnp.float32

    def blk(pfx):
        return (p[pfx + 'c3_W'],
                p[pfx + 'c5_W'],
                p[pfx + 'dw_W'].reshape(C, 3),
                p[pfx + 'c3_b'].reshape(1, C), p[pfx + 'c5_b'].reshape(1, C),
                p[pfx + 'bn1_g'].reshape(1, C), p[pfx + 'bn1_b'].reshape(1, C),
                p[pfx + 'bn2_g'].reshape(1, C), p[pfx + 'bn2_b'].reshape(1, C),
                p[pfx + 'dw_b'].reshape(1, C),
                p[pfx + 'pw_W'].reshape(C, C),
                p[pfx + 'pw_b'].reshape(1, C),
                p[pfx + 'bn3_g'].reshape(1, C), p[pfx + 'bn3_b'].reshape(1, C),
                p[pfx + 'fus_W'].reshape(C, 2 * C),
                p[pfx + 'fus_b'].reshape(1, C))

    args = ((data.astype(f32), p['emb'].astype(f32),
             p['att_i'].reshape(1, 2 * C), p['att_j'].reshape(1, 2 * C),
             p['lin_W'].reshape(1, C), p['gnn_bias'].reshape(1, C))
            + blk('tcn1_') + blk('tcn2_')
            + (p['bno_g'].reshape(1, C), p['bno_b'].reshape(1, C),
               p['out_W'], p['out_b'].reshape(1, 1)))

    return pl.pallas_call(
        _body,
        out_shape=jax.ShapeDtypeStruct((B, NN), f32),
    )(*args)


# speedup vs baseline: 1.4729x; 1.4729x over previous
import jax
import jax.numpy as jnp
from jax.experimental import pallas as pl

NN = 128
C = 64
T = 64
B = 16
TOPK = 20
TACT = 16
NROWS = 136
BG_W = 240.0
BN_CNT = 2048.0 * T


def _mm(a, b):
    sh = a.shape
    a2 = a.reshape(-1, sh[-1])
    if a2.dtype != jnp.bfloat16:
        a2 = a2.astype(jnp.bfloat16)
    r = jax.lax.dot_general(a2, b.astype(jnp.bfloat16), (((1,), (0,)), ((), ())),
                            preferred_element_type=jnp.float32)
    return r.reshape(sh[:-1] + (b.shape[-1],))


def _mmt(a, b):
    sh = a.shape
    a2 = a.reshape(-1, sh[-1])
    if a2.dtype != jnp.bfloat16:
        a2 = a2.astype(jnp.bfloat16)
    r = jax.lax.dot_general(a2, b.astype(jnp.bfloat16), (((1,), (1,)), ((), ())),
                            preferred_element_type=jnp.float32)
    return r.reshape(sh[:-1] + (b.shape[0],))


def _tshift(x, s):
    if s == 0:
        return x
    r, t, c = x.shape
    z = jnp.zeros((r, abs(s), c), x.dtype)
    if s > 0:
        return jnp.concatenate([x[:, s:, :], z], axis=1)
    return jnp.concatenate([z, x[:, :t + s, :]], axis=1)


def _body(data_ref, emb_ref, atti_ref, attj_ref, lin_ref, gbias_ref,
          w3t0, w5t0, dwt0, c3b0, c5b0, bn1g0, bn1b0, bn2g0, bn2b0,
          dwb0, pww0, pwb0, bn3g0, bn3b0, fww0, fusb0,
          w3t1, w5t1, dwt1, c3b1, c5b1, bn1g1, bn1b1, bn2g1, bn2b1,
          dwb1, pww1, pwb1, bn3g1, bn3b1, fww1, fusb1,
          bnog_ref, bnob_ref, outw_ref, outb_ref, out_ref):
    f32 = jnp.float32
    emb = emb_ref[...]
    linv = lin_ref[...]

    ebf = emb.astype(jnp.bfloat16)
    g = jax.lax.dot_general(ebf, ebf.T, (((1,), (0,)), ((), ())),
                            preferred_element_type=jnp.float32)
    nrm = jnp.sqrt(jnp.sum(emb * emb, axis=1, keepdims=True))
    cos = g / (nrm * nrm.T)
    mask = jnp.zeros((NN, NN), f32)
    cm = cos
    for _ in range(TOPK):
        rmax = jnp.max(cm, axis=1, keepdims=True)
        hit = cm == rmax
        mask = jnp.where(hit, 1.0, mask)
        cm = jnp.where(hit, -jnp.inf, cm)

    att_i = atti_ref[...]
    att_j = attj_ref[...]
    a_i = jnp.sum(linv * att_i[:, :C])
    a_j = jnp.sum(linv * att_j[:, :C])
    e_i = jnp.sum(emb * att_i[:, C:], axis=1, keepdims=True)
    e_j = jnp.sum(emb * att_j[:, C:], axis=1, keepdims=True)

    xa = data_ref[0, :, 0:TACT]
    xt = xa.T
    alpha = (xt[:, :, None] * a_i + e_i.reshape(1, NN, 1)
             + xt[:, None, :] * a_j + e_j.reshape(1, 1, NN))
    alpha = jnp.where(alpha >= 0, alpha, 0.2 * alpha)
    am = jnp.where(mask[None, :, :] > 0, alpha, -jnp.inf)
    amax = jnp.max(am, axis=2, keepdims=True)
    ex = jnp.exp(am - amax)
    denom = jnp.sum(ex, axis=2, keepdims=True) + 1e-16
    att = ex / denom
    s = jnp.sum(att * xt[:, None, :], axis=2)

    gnn_bias = gbias_ref[...].reshape(1, 1, C)
    st3 = s.T.reshape(NN, TACT, 1)
    x_act = st3 * linv.reshape(1, 1, C) + gnn_bias
    x_all = jnp.concatenate(
        [jnp.concatenate(
            [x_act,
             jnp.broadcast_to(gnn_bias, (NN, T - TACT, C))], axis=1),
         jnp.broadcast_to(gnn_bias, (NROWS - NN, T, C))], axis=0)

    ridx = jax.lax.broadcasted_iota(jnp.int32, (NROWS, 1, 1), 0)
    wr = jnp.where(ridx < NN, 1.0, BG_W)

    def bn(x, gv, bv):
        xw = x * wr
        m = jnp.sum(xw, axis=(0, 1), keepdims=True) / BN_CNT
        msq = jnp.sum(xw * x, axis=(0, 1), keepdims=True) / BN_CNT
        v = msq - m * m
        sc = gv / jnp.sqrt(v + 1e-5)
        return x * sc + (bv - m * sc)

    def row3(ref):
        return ref[...].reshape(1, 1, -1)

    zoo = jnp.zeros((C, C), jnp.bfloat16)

    def ms_block(x, w3t, w5t, dwt, c3b, c5b, bn1g, bn1b, bn2g, bn2b,
                 dwb, pwt, pwb, bn3g, bn3b, fww, fusb):
        xb = x.astype(jnp.bfloat16)
        xcat = jnp.concatenate(
            [_tshift(xb, d) for d in (-2, -1, 0, 1, 2)], axis=2)
        w3b = w3t[...].astype(jnp.bfloat16)
        w5b = w5t[...].astype(jnp.bfloat16)
        pwtv = pwt[...]
        rhs = jnp.concatenate(
            [jnp.concatenate(
                [w3b[dk - 1] if 1 <= dk <= 3 else zoo,
                 w5b[dk],
                 (pwtv * dwt[dk - 1].reshape(C, 1)).astype(jnp.bfloat16)
                 if 1 <= dk <= 3 else zoo], axis=1)
             for dk in range(5)], axis=0)
        acc = _mm(xcat, rhs)
        bias_res = _mm(dwb[...], pwtv) + pwb[...]
        res = acc[:, :, 2 * C:] + bias_res.reshape(1, 1, C)
        res = bn(res, row3(bn3g), row3(bn3b))
        b12 = acc[:, :, :2 * C] + jnp.concatenate(
            [row3(c3b), row3(c5b)], axis=2)
        b12 = jax.nn.relu(bn(b12,
                             jnp.concatenate([row3(bn1g), row3(bn2g)], axis=2),
                             jnp.concatenate([row3(bn1b), row3(bn2b)], axis=2)))
        fused = _mmt(b12, fww[...]) + row3(fusb)
        return jax.nn.relu(fused + res)

    h = ms_block(x_all, w3t0, w5t0, dwt0, c3b0, c5b0, bn1g0, bn1b0,
                 bn2g0, bn2b0, dwb0, pww0, pwb0, bn3g0, bn3b0, fww0, fusb0)
    h = ms_block(h, w3t1, w5t1, dwt1, c3b1, c5b1, bn1g1, bn1b1,
                 bn2g1, bn2b1, dwb1, pww1, pwb1, bn3g1, bn3b1, fww1, fusb1)
    hm = jnp.sum(h, axis=1) / float(T)

    y0 = hm[:NN, :] * emb
    ybg = hm[NN:NN + 1, :] * emb
    m = jnp.sum(y0 + 15.0 * ybg, axis=0, keepdims=True) / 2048.0
    d0 = y0 - m
    dbg = ybg - m
    v = (jnp.sum(d0 * d0, axis=0, keepdims=True)
         + 15.0 * jnp.sum(dbg * dbg, axis=0, keepdims=True)) / 2048.0
    bno_g = bnog_ref[...]
    bno_b = bnob_ref[...]
    z0 = jax.nn.relu(d0 / jnp.sqrt(v + 1e-5) * bno_g + bno_b)
    zbg = jax.nn.relu(dbg / jnp.sqrt(v + 1e-5) * bno_g + bno_b)
    outwT = outw_ref[...].T
    row0 = (_mm(z0, outwT) + outb_ref[0, 0]).T
    rbg = (_mm(zbg, outwT) + outb_ref[0, 0]).T
    out_ref[...] = jnp.concatenate(
        [row0, jnp.broadcast_to(rbg, (B - 1, NN))], axis=0)


def kernel(data, params):
    p = params
    f32 = jnp.float32

    def blk(pfx):
        return (p[pfx + 'c3_W'].transpose(2, 1, 0),
                p[pfx + 'c5_W'].transpose(2, 1, 0),
                p[pfx + 'dw_W'][:, 0, :].T,
                p[pfx + 'c3_b'].reshape(1, C), p[pfx + 'c5_b'].reshape(1, C),
                p[pfx + 'bn1_g'].reshape(1, C), p[pfx + 'bn1_b'].reshape(1, C),
                p[pfx + 'bn2_g'].reshape(1, C), p[pfx + 'bn2_b'].reshape(1, C),
                p[pfx + 'dw_b'].reshape(1, C),
                p[pfx + 'pw_W'][:, :, 0].T,
                p[pfx + 'pw_b'].reshape(1, C),
                p[pfx + 'bn3_g'].reshape(1, C), p[pfx + 'bn3_b'].reshape(1, C),
                p[pfx + 'fus_W'].reshape(C, 2 * C),
                p[pfx + 'fus_b'].reshape(1, C))

    args = ((data.astype(f32), p['emb'].astype(f32),
             p['att_i'].reshape(1, 2 * C), p['att_j'].reshape(1, 2 * C),
             p['lin_W'].reshape(1, C), p['gnn_bias'].reshape(1, C))
            + blk('tcn1_') + blk('tcn2_')
            + (p['bno_g'].reshape(1, C), p['bno_b'].reshape(1, C),
               p['out_W'], p['out_b'].reshape(1, 1)))

    return pl.pallas_call(
        _body,
        out_shape=jax.ShapeDtypeStruct((B, NN), f32),
    )(*args)
